# baseline (speedup 1.0000x reference)
import functools

import jax
import jax.numpy as jnp
from jax import lax
from jax.experimental import pallas as pl
from jax.experimental.pallas import tpu as pltpu
from jax.experimental.pallas import tpu_sc as plsc

BATCH = 16384
FEAT = 16
PACK = 128 // FEAT

_DN_T = (((1,), (1,)), ((), ()))


def _sc_gather(table, idx):
    n = idx.shape[0]
    info = plsc.get_sparse_core_info()
    nw = info.num_cores * info.num_subcores
    b_per_w = n // nw
    mesh = plsc.VectorSubcoreMesh(core_axis_name="c", subcore_axis_name="s")
    half = b_per_w // 2

    @functools.partial(
        pl.kernel,
        mesh=mesh,
        compiler_params=pltpu.CompilerParams(use_tc_tiling_on_sc=False),
        out_type=jax.ShapeDtypeStruct((n, FEAT), jnp.float32),
        scratch_types=[
            pltpu.VMEM((half,), jnp.int32),
            pltpu.VMEM((half,), jnp.int32),
            pltpu.VMEM((half, FEAT), jnp.float32),
            pltpu.VMEM((half, FEAT), jnp.float32),
            pltpu.SemaphoreType.DMA,
            pltpu.SemaphoreType.DMA,
            pltpu.SemaphoreType.DMA,
            pltpu.SemaphoreType.DMA,
        ],
    )
    def gather_k(table_hbm, idx_hbm, out_hbm, idx_v0, idx_v1, rows_v0,
                 rows_v1, si0, si1, sg0, sg1):
        wid = lax.axis_index("s") * info.num_cores + lax.axis_index("c")
        base = wid * b_per_w
        i0 = pltpu.async_copy(idx_hbm.at[pl.ds(base, half)], idx_v0, si0)
        i1 = pltpu.async_copy(idx_hbm.at[pl.ds(base + half, half)], idx_v1,
                              si1)
        i0.wait()
        g0 = pltpu.async_copy(table_hbm.at[idx_v0], rows_v0, sg0)
        i1.wait()
        g1 = pltpu.async_copy(table_hbm.at[idx_v1], rows_v1, sg1)
        g0.wait()
        w0 = pltpu.async_copy(rows_v0, out_hbm.at[pl.ds(base, half)], si0)
        g1.wait()
        w1 = pltpu.async_copy(rows_v1, out_hbm.at[pl.ds(base + half, half)],
                              si1)
        w0.wait()
        w1.wait()

    return gather_k(table, idx)


def _blockdiag(w_tiled, rows, cols, rblk, cblk):
    r = lax.broadcasted_iota(jnp.int32, (rows, cols), 0)
    c = lax.broadcasted_iota(jnp.int32, (rows, cols), 1)
    return jnp.where((r // rblk) == (c // cblk), w_tiled, 0.0)


def _tc_mlp(xp, igp, w1, b1, w2, b2, w3, b3):
    rows = xp.shape[0]
    blk = min(rows, 1024)
    grid = rows // blk
    f32 = jnp.float32

    def body(x_ref, ig_ref, w1_ref, b1_ref, w2_ref, b2_ref, w3_ref, b3_ref,
             o_ref):
        w1t = jnp.concatenate(
            [jnp.zeros((32, 2), f32), w1_ref[:], jnp.zeros((32, 5), f32)],
            axis=1)
        w1k = _blockdiag(jnp.tile(w1t, (8, 8)), 256, 128, 32, 16)
        w2k = _blockdiag(jnp.tile(w2_ref[:], (8, 8)), 128, 256, 16, 32)
        w3k = _blockdiag(jnp.tile(w3_ref[:], (8, 8)), 16, 128, 2, 16)
        sr = lax.broadcasted_iota(jnp.int32, (16, 128), 0)
        sc = lax.broadcasted_iota(jnp.int32, (16, 128), 1)
        selk = jnp.where(sc == 16 * (sr // 2) + (sr % 2), 1.0, 0.0)
        b1k = jnp.tile(b1_ref[:], (1, 8))
        b2k = jnp.tile(b2_ref[:], (1, 8))
        b3k = jnp.tile(b3_ref[:], (1, 8))

        x = x_ref[:] + ig_ref[:]
        h = lax.dot_general(x, w1k, _DN_T, preferred_element_type=f32)
        h = jnp.maximum(h + b1k, 0.0)
        h = lax.dot_general(h, w2k, _DN_T, preferred_element_type=f32)
        h = jnp.maximum(h + b2k, 0.0)
        r = lax.dot_general(h, w3k, _DN_T, preferred_element_type=f32)
        base = lax.dot_general(x, selk, _DN_T, preferred_element_type=f32)
        out = base + r + b3k
        o_ref[:] = out.reshape(blk, PACK, 2)

    full = lambda shape: pl.BlockSpec(shape, lambda i: (0, 0))
    return pl.pallas_call(
        body,
        grid=(grid,),
        in_specs=[
            pl.BlockSpec((blk, 128), lambda i: (i, 0)),
            pl.BlockSpec((blk, 128), lambda i: (i, 0)),
            full((32, 9)),
            full((1, 32)),
            full((16, 32)),
            full((1, 16)),
            full((2, 16)),
            full((1, 2)),
        ],
        out_specs=pl.BlockSpec((blk, PACK, 2), lambda i: (i, 0, 0)),
        out_shape=jax.ShapeDtypeStruct((rows, PACK, 2), jnp.float32),
    )(xp, igp, w1, b1, w2, b2, w3, b3)


def kernel(action_idx, is_ground, physics_params, action_emb, W1, b1, W2, b2,
           W3, b3, gravity):
    idx = action_idx.astype(jnp.int32)
    n = physics_params.shape[0]
    f32 = jnp.float32
    table = jnp.concatenate(
        [physics_params[:, :2], action_emb, jnp.zeros((n, FEAT - 10), f32)],
        axis=1)
    onehot10 = jnp.zeros((1, 1, FEAT), f32).at[0, 0, 10].set(1.0)
    igp = (is_ground.reshape(BATCH // PACK, PACK, 1) * onehot10).reshape(
        BATCH // PACK, 128)

    halfb = BATCH // 2
    halfp = halfb // PACK
    w1r, b1r = W1, b1.reshape(1, 32)
    w2r, b2r = W2, b2.reshape(1, 16)
    w3r, b3r = W3, b3.reshape(1, 2)
    xg1 = _sc_gather(table, idx[:halfb])
    xg2 = _sc_gather(table, idx[halfb:])
    out1 = _tc_mlp(xg1.reshape(halfp, 128), igp[:halfp], w1r, b1r, w2r, b2r,
                   w3r, b3r)
    out2 = _tc_mlp(xg2.reshape(halfp, 128), igp[halfp:], w1r, b1r, w2r, b2r,
                   w3r, b3r)
    outp = jnp.concatenate([out1, out2], axis=0)
    return (outp.reshape(BATCH, 2), gravity)


# speedup vs baseline: 1.3278x; 1.3278x over previous
import functools

import jax
import jax.numpy as jnp
from jax import lax
from jax.experimental import pallas as pl
from jax.experimental.pallas import tpu as pltpu
from jax.experimental.pallas import tpu_sc as plsc

BATCH = 16384
FEAT = 16
PACK = 128 // FEAT

_DN_T = (((1,), (1,)), ((), ()))


def _sc_gather(table, idx):
    info = plsc.get_sparse_core_info()
    nw = info.num_cores * info.num_subcores
    b_per_w = BATCH // nw
    mesh = plsc.VectorSubcoreMesh(core_axis_name="c", subcore_axis_name="s")
    half = b_per_w // 2

    @functools.partial(
        pl.kernel,
        mesh=mesh,
        compiler_params=pltpu.CompilerParams(use_tc_tiling_on_sc=False),
        out_type=jax.ShapeDtypeStruct((BATCH, FEAT), jnp.float32),
        scratch_types=[
            pltpu.VMEM((half,), jnp.int32),
            pltpu.VMEM((half,), jnp.int32),
            pltpu.VMEM((half, FEAT), jnp.float32),
            pltpu.VMEM((half, FEAT), jnp.float32),
            pltpu.SemaphoreType.DMA,
            pltpu.SemaphoreType.DMA,
            pltpu.SemaphoreType.DMA,
            pltpu.SemaphoreType.DMA,
        ],
    )
    def gather_k(table_hbm, idx_hbm, out_hbm, idx_v0, idx_v1, rows_v0,
                 rows_v1, si0, si1, sg0, sg1):
        wid = lax.axis_index("s") * info.num_cores + lax.axis_index("c")
        base = wid * b_per_w
        i0 = pltpu.async_copy(idx_hbm.at[pl.ds(base, half)], idx_v0, si0)
        i1 = pltpu.async_copy(idx_hbm.at[pl.ds(base + half, half)], idx_v1,
                              si1)
        i0.wait()
        g0 = pltpu.async_copy(table_hbm.at[idx_v0], rows_v0, sg0)
        i1.wait()
        g1 = pltpu.async_copy(table_hbm.at[idx_v1], rows_v1, sg1)
        g0.wait()
        w0 = pltpu.async_copy(rows_v0, out_hbm.at[pl.ds(base, half)], si0)
        g1.wait()
        w1 = pltpu.async_copy(rows_v1, out_hbm.at[pl.ds(base + half, half)],
                              si1)
        w0.wait()
        w1.wait()

    return gather_k(table, idx)


def _blockdiag(w_tiled, rows, cols, rblk, cblk):
    r = lax.broadcasted_iota(jnp.int32, (rows, cols), 0)
    c = lax.broadcasted_iota(jnp.int32, (rows, cols), 1)
    return jnp.where((r // rblk) == (c // cblk), w_tiled, 0.0)


def _tc_mlp(xp, igp, w1, b1, w2, b2, w3, b3):
    rows = BATCH // PACK
    blk = 1024
    grid = rows // blk
    f32 = jnp.float32

    def body(x_ref, ig_ref, w1_ref, b1_ref, w2_ref, b2_ref, w3_ref, b3_ref,
             o_ref):
        w1t = jnp.concatenate(
            [jnp.zeros((32, 2), f32), w1_ref[:], jnp.zeros((32, 5), f32)],
            axis=1)
        w1k = _blockdiag(jnp.tile(w1t, (8, 8)), 256, 128, 32, 16)
        w2k = _blockdiag(jnp.tile(w2_ref[:], (8, 8)), 128, 256, 16, 32)
        w3k = _blockdiag(jnp.tile(w3_ref[:], (8, 8)), 16, 128, 2, 16)
        sr = lax.broadcasted_iota(jnp.int32, (16, 128), 0)
        sc = lax.broadcasted_iota(jnp.int32, (16, 128), 1)
        selk = jnp.where(sc == 16 * (sr // 2) + (sr % 2), 1.0, 0.0)
        b1k = jnp.tile(b1_ref[:], (1, 8))
        b2k = jnp.tile(b2_ref[:], (1, 8))
        b3k = jnp.tile(b3_ref[:], (1, 8))

        x = x_ref[:] + ig_ref[:]
        h = lax.dot_general(x, w1k, _DN_T, preferred_element_type=f32)
        h = jnp.maximum(h + b1k, 0.0)
        h = lax.dot_general(h, w2k, _DN_T, preferred_element_type=f32)
        h = jnp.maximum(h + b2k, 0.0)
        r = lax.dot_general(h, w3k, _DN_T, preferred_element_type=f32)
        base = lax.dot_general(x, selk, _DN_T, preferred_element_type=f32)
        out = base + r + b3k
        o_ref[:] = out.reshape(blk, PACK, 2)

    full = lambda shape: pl.BlockSpec(shape, lambda i: (0, 0))
    return pl.pallas_call(
        body,
        grid=(grid,),
        in_specs=[
            pl.BlockSpec((blk, 128), lambda i: (i, 0)),
            pl.BlockSpec((blk, 128), lambda i: (i, 0)),
            full((32, 9)),
            full((1, 32)),
            full((16, 32)),
            full((1, 16)),
            full((2, 16)),
            full((1, 2)),
        ],
        out_specs=pl.BlockSpec((blk, PACK, 2), lambda i: (i, 0, 0)),
        out_shape=jax.ShapeDtypeStruct((rows, PACK, 2), jnp.float32),
    )(xp, igp, w1, b1, w2, b2, w3, b3)


def kernel(action_idx, is_ground, physics_params, action_emb, W1, b1, W2, b2,
           W3, b3, gravity):
    idx = action_idx.astype(jnp.int32)
    n = physics_params.shape[0]
    f32 = jnp.float32
    table = jnp.concatenate(
        [physics_params[:, :2], action_emb, jnp.zeros((n, FEAT - 10), f32)],
        axis=1)
    onehot10 = jnp.zeros((1, 1, FEAT), f32).at[0, 0, 10].set(1.0)
    igp = (is_ground.reshape(BATCH // PACK, PACK, 1) * onehot10).reshape(
        BATCH // PACK, 128)

    xg = _sc_gather(table, idx)
    xp = xg.reshape(BATCH // PACK, 128)
    outp = _tc_mlp(xp, igp, W1, b1.reshape(1, 32), W2, b2.reshape(1, 16),
                   W3, b3.reshape(1, 2))
    return (outp.reshape(BATCH, 2), gravity)


# speedup vs baseline: 1.3350x; 1.0054x over previous
import functools

import jax
import jax.numpy as jnp
from jax import lax
from jax.experimental import pallas as pl
from jax.experimental.pallas import tpu as pltpu
from jax.experimental.pallas import tpu_sc as plsc

BATCH = 16384
FEAT = 16
PACK = 128 // FEAT

_DN_T = (((1,), (1,)), ((), ()))


def _sc_gather(table, idx):
    info = plsc.get_sparse_core_info()
    nw = info.num_cores * info.num_subcores
    b_per_w = BATCH // nw
    mesh = plsc.VectorSubcoreMesh(core_axis_name="c", subcore_axis_name="s")
    nst = 4
    q = b_per_w // nst

    @functools.partial(
        pl.kernel,
        mesh=mesh,
        compiler_params=pltpu.CompilerParams(use_tc_tiling_on_sc=False),
        out_type=jax.ShapeDtypeStruct((BATCH, FEAT), jnp.float32),
        scratch_types=(
            [pltpu.VMEM((q,), jnp.int32)] * nst
            + [pltpu.VMEM((q, FEAT), jnp.float32)] * nst
            + [pltpu.SemaphoreType.DMA] * (2 * nst)
        ),
    )
    def gather_k(table_hbm, idx_hbm, out_hbm, *scratch):
        idx_v = scratch[:nst]
        rows_v = scratch[nst:2 * nst]
        si = scratch[2 * nst:3 * nst]
        sg = scratch[3 * nst:]
        wid = lax.axis_index("s") * info.num_cores + lax.axis_index("c")
        base = wid * b_per_w
        icopies = [
            pltpu.async_copy(idx_hbm.at[pl.ds(base + k * q, q)], idx_v[k],
                             si[k]) for k in range(nst)
        ]
        gathers = []
        for k in range(nst):
            icopies[k].wait()
            gathers.append(
                pltpu.async_copy(table_hbm.at[idx_v[k]], rows_v[k], sg[k]))
        writes = []
        for k in range(nst):
            gathers[k].wait()
            writes.append(
                pltpu.async_copy(rows_v[k], out_hbm.at[pl.ds(base + k * q, q)],
                                 si[k]))
        for w in writes:
            w.wait()

    return gather_k(table, idx)


def _blockdiag(w_tiled, rows, cols, rblk, cblk):
    r = lax.broadcasted_iota(jnp.int32, (rows, cols), 0)
    c = lax.broadcasted_iota(jnp.int32, (rows, cols), 1)
    return jnp.where((r // rblk) == (c // cblk), w_tiled, 0.0)


def _tc_mlp(xp, igp, w1, b1, w2, b2, w3, b3):
    rows = BATCH // PACK
    blk = 1024
    grid = rows // blk
    f32 = jnp.float32

    def body(x_ref, ig_ref, w1_ref, b1_ref, w2_ref, b2_ref, w3_ref, b3_ref,
             o_ref):
        w1t = jnp.concatenate(
            [jnp.zeros((32, 2), f32), w1_ref[:], jnp.zeros((32, 5), f32)],
            axis=1)
        w1k = _blockdiag(jnp.tile(w1t, (8, 8)), 256, 128, 32, 16)
        w2k = _blockdiag(jnp.tile(w2_ref[:], (8, 8)), 128, 256, 16, 32)
        w3k = _blockdiag(jnp.tile(w3_ref[:], (8, 8)), 16, 128, 2, 16)
        sr = lax.broadcasted_iota(jnp.int32, (16, 128), 0)
        sc = lax.broadcasted_iota(jnp.int32, (16, 128), 1)
        selk = jnp.where(sc == 16 * (sr // 2) + (sr % 2), 1.0, 0.0)
        b1k = jnp.tile(b1_ref[:], (1, 8))
        b2k = jnp.tile(b2_ref[:], (1, 8))
        b3k = jnp.tile(b3_ref[:], (1, 8))

        x = x_ref[:] + ig_ref[:]
        h = lax.dot_general(x, w1k, _DN_T, preferred_element_type=f32)
        h = jnp.maximum(h + b1k, 0.0)
        h = lax.dot_general(h, w2k, _DN_T, preferred_element_type=f32)
        h = jnp.maximum(h + b2k, 0.0)
        r = lax.dot_general(h, w3k, _DN_T, preferred_element_type=f32)
        base = lax.dot_general(x, selk, _DN_T, preferred_element_type=f32)
        out = base + r + b3k
        o_ref[:] = out.reshape(blk, PACK, 2)

    full = lambda shape: pl.BlockSpec(shape, lambda i: (0, 0))
    return pl.pallas_call(
        body,
        grid=(grid,),
        in_specs=[
            pl.BlockSpec((blk, 128), lambda i: (i, 0)),
            pl.BlockSpec((blk, 128), lambda i: (i, 0)),
            full((32, 9)),
            full((1, 32)),
            full((16, 32)),
            full((1, 16)),
            full((2, 16)),
            full((1, 2)),
        ],
        out_specs=pl.BlockSpec((blk, PACK, 2), lambda i: (i, 0, 0)),
        out_shape=jax.ShapeDtypeStruct((rows, PACK, 2), jnp.float32),
    )(xp, igp, w1, b1, w2, b2, w3, b3)


def kernel(action_idx, is_ground, physics_params, action_emb, W1, b1, W2, b2,
           W3, b3, gravity):
    idx = action_idx.astype(jnp.int32)
    n = physics_params.shape[0]
    f32 = jnp.float32
    table = jnp.concatenate(
        [physics_params[:, :2], action_emb, jnp.zeros((n, FEAT - 10), f32)],
        axis=1)
    onehot10 = jnp.zeros((1, 1, FEAT), f32).at[0, 0, 10].set(1.0)
    igp = (is_ground.reshape(BATCH // PACK, PACK, 1) * onehot10).reshape(
        BATCH // PACK, 128)

    xg = _sc_gather(table, idx)
    xp = xg.reshape(BATCH // PACK, 128)
    outp = _tc_mlp(xp, igp, W1, b1.reshape(1, 32), W2, b2.reshape(1, 16),
                   W3, b3.reshape(1, 2))
    return (outp.reshape(BATCH, 2), gravity)
